# Initial kernel scaffold; baseline (speedup 1.0000x reference)
import jax
import jax.numpy as jnp
from jax.experimental import pallas as pl


def kernel(x, y, memory, idx):
    raise NotImplementedError("write your pallas kernel here")


# speedup vs baseline: 5.5293x; 5.5293x over previous
import functools
import jax
import jax.numpy as jnp
from jax import lax
from jax.experimental import pallas as pl
from jax.experimental.pallas import tpu as pltpu

B = 256
D = 128
OUT_SIZE = 100000
K = 512
T = 0.07
NDRAW = 4097
VPAD = 100352
NV = 2048


def _mm_body(x_ref, m_ref, o_ref):
    o_ref[...] = lax.dot_general(
        x_ref[...], m_ref[...], (((1,), (1,)), ((), ())),
        preferred_element_type=jnp.float32)


def _scores(x, mem_pad):
    return pl.pallas_call(
        _mm_body,
        grid=(VPAD // NV,),
        in_specs=[
            pl.BlockSpec((B, D), lambda i: (0, 0)),
            pl.BlockSpec((NV, D), lambda i: (i, 0)),
        ],
        out_specs=pl.BlockSpec((B, NV), lambda i: (0, i)),
        out_shape=jax.ShapeDtypeStruct((B, VPAD), jnp.float32),
    )(x, mem_pad)


def kernel(x, y, memory, idx):
    mem_pad = jnp.concatenate(
        [memory, jnp.zeros((VPAD - OUT_SIZE, D), jnp.float32)], axis=0)
    S = _scores(x, mem_pad)
    idx2 = jnp.concatenate([y[:, None], idx[:, 1:]], axis=1)
    s = jnp.take_along_axis(S, idx2, axis=1)
    out = jnp.exp(s / T)
    srt = jnp.sort(out[:, 1:], axis=1)[:, ::-1]
    out_sel = jnp.concatenate(
        [out[:, :1], srt[:, :K // 2], srt[:, 4096 - K:4096 - K + K // 2]], axis=1)
    Z = jnp.mean(out_sel) * OUT_SIZE
    return out_sel / Z


# speedup vs baseline: 5.7017x; 1.0312x over previous
import functools
import jax
import jax.numpy as jnp
from jax import lax
from jax.experimental import pallas as pl
from jax.experimental.pallas import tpu as pltpu
from jax.experimental.pallas import tpu_sc as plsc

B = 256
D = 128
OUT_SIZE = 100000
K = 512
T = 0.07
NDRAW = 4097
VPAD = 100352
NV = 2048

NC = 2
NS = 16
NW = NC * NS
BPW = B // NW
NIDX = 4112


def _mm_body(x_ref, m_ref, o_ref):
    o_ref[...] = lax.dot_general(
        x_ref[...], m_ref[...], (((1,), (1,)), ((), ())),
        preferred_element_type=jnp.float32)


def _scores(x, mem_pad):
    return pl.pallas_call(
        _mm_body,
        grid=(VPAD // NV,),
        in_specs=[
            pl.BlockSpec((B, D), lambda i: (0, 0)),
            pl.BlockSpec((NV, D), lambda i: (i, 0)),
        ],
        out_specs=pl.BlockSpec((B, NV), lambda i: (0, i)),
        out_shape=jax.ShapeDtypeStruct((B, VPAD), jnp.float32),
    )(x, mem_pad)


_sc_mesh = plsc.VectorSubcoreMesh(core_axis_name="c", subcore_axis_name="s")


@functools.partial(
    pl.kernel,
    out_type=jax.ShapeDtypeStruct((B, NIDX), jnp.float32),
    mesh=_sc_mesh,
    scratch_types=[
        pltpu.VMEM((VPAD,), jnp.float32),
        pltpu.VMEM((NIDX,), jnp.int32),
        pltpu.VMEM((NIDX,), jnp.float32),
    ],
    compiler_params=pltpu.CompilerParams(needs_layout_passes=False),
)
def _sc_gather(s_hbm, idx_hbm, out_hbm, srow_v, idx_v, out_v):
    wid = lax.axis_index("s") * NC + lax.axis_index("c")

    def batch_body(i, carry):
        b = wid * BPW + i
        pltpu.sync_copy(s_hbm.at[b], srow_v)
        pltpu.sync_copy(idx_hbm.at[b], idx_v)

        def g_body(j, carry2):
            ii = idx_v[pl.ds(j * 16, 16)]
            out_v[pl.ds(j * 16, 16)] = plsc.load_gather(srow_v, [ii])
            return carry2

        lax.fori_loop(0, NIDX // 16, g_body, 0, unroll=4)
        pltpu.sync_copy(out_v, out_hbm.at[b])
        return carry

    lax.fori_loop(0, BPW, batch_body, 0)


def kernel(x, y, memory, idx):
    mem_pad = jnp.concatenate(
        [memory, jnp.zeros((VPAD - OUT_SIZE, D), jnp.float32)], axis=0)
    S = _scores(x, mem_pad)
    idx_pad = jnp.concatenate(
        [y[:, None], idx[:, 1:],
         jnp.zeros((B, NIDX - NDRAW), jnp.int32)], axis=1)
    s = _sc_gather(S, idx_pad)[:, :NDRAW]
    out = jnp.exp(s / T)
    srt = jnp.sort(out[:, 1:], axis=1)[:, ::-1]
    out_sel = jnp.concatenate(
        [out[:, :1], srt[:, :K // 2], srt[:, 4096 - K:4096 - K + K // 2]], axis=1)
    Z = jnp.mean(out_sel) * OUT_SIZE
    return out_sel / Z


# speedup vs baseline: 8.5493x; 1.4994x over previous
import functools
import jax
import jax.numpy as jnp
import numpy as np
from jax import lax
from jax.experimental import pallas as pl
from jax.experimental.pallas import tpu as pltpu
from jax.experimental.pallas import tpu_sc as plsc

B = 256
D = 128
OUT_SIZE = 100000
K = 512
T = 0.07
NDRAW = 4097
N = 4096
VPAD = 100352
NV = 2048

NC = 2
NS = 16
NW = NC * NS
BPW = B // NW
NIDX = 4112
NOUT = 528

_U32 = jnp.uint32
_TOPBIT = np.uint32(0x80000000)
_ALLBIT = np.uint32(0xFFFFFFFF)


def _mm_body(x_ref, m_ref, o_ref):
    o_ref[...] = lax.dot_general(
        x_ref[...], m_ref[...], (((1,), (1,)), ((), ())),
        preferred_element_type=jnp.float32)


def _scores(x, mem_pad):
    return pl.pallas_call(
        _mm_body,
        grid=(VPAD // NV,),
        in_specs=[
            pl.BlockSpec((B, D), lambda i: (0, 0)),
            pl.BlockSpec((NV, D), lambda i: (i, 0)),
        ],
        out_specs=pl.BlockSpec((B, NV), lambda i: (0, i)),
        out_shape=jax.ShapeDtypeStruct((B, VPAD), jnp.float32),
    )(x, mem_pad)


def _iota16():
    return lax.iota(jnp.int32, 16)


def _splat(v, dtype=jnp.int32):
    return jnp.full((16,), v, dtype)


def _keyify(v):
    b = plsc.bitcast(v, _U32)
    neg = b >= _TOPBIT
    return jnp.where(neg, b ^ _ALLBIT, b | _TOPBIT)


def _unkey(k):
    b = jnp.where(k >= _TOPBIT, k ^ _TOPBIT, k ^ _ALLBIT)
    return plsc.bitcast(b, jnp.float32)


def _popcnt(mask):
    return plsc.all_reduce_population_count(mask)


def _ldu(ref, start):
    return plsc.bitcast(ref[pl.ds(start, 16)], _U32)


def _stu(ref, start, v):
    ref[pl.ds(start, 16)] = plsc.bitcast(v, jnp.int32)


def _gathu(ref, idxv):
    return plsc.bitcast(plsc.load_gather(ref, [idxv]), _U32)


def _floop(n, body):
    lax.fori_loop(0, n, lambda i, c: (body(i), 0)[1], 0)


def _sort_desc(buf, nvec):
    def presort(i):
        k = _ldu(buf, i * 16)
        _stu(buf, i * 16, plsc.sort_key_val(k, k, descending=True)[0])
    _floop(nvec, presort)
    run = 1
    while run < nvec:
        npairs = nvec // (2 * run)
        r = run

        def merge(p, r=r):
            base = p * (2 * r) * 16
            for i in range(r):
                lo_i = base + i * 16
                hi_i = base + (2 * r - 1 - i) * 16
                a = _ldu(buf, lo_i)
                brev = lax.rev(_ldu(buf, hi_i), (0,))
                _stu(buf, lo_i, jnp.maximum(a, brev))
                _stu(buf, hi_i, lax.rev(jnp.minimum(a, brev), (0,)))
            for half in range(2):
                hbase = base + half * r * 16
                dist = r // 2
                while dist >= 1:
                    for g in range(0, r, 2 * dist):
                        for i in range(dist):
                            i0 = hbase + (g + i) * 16
                            i1 = hbase + (g + i + dist) * 16
                            a = _ldu(buf, i0)
                            b2 = _ldu(buf, i1)
                            _stu(buf, i0, jnp.maximum(a, b2))
                            _stu(buf, i1, jnp.minimum(a, b2))
                    dist //= 2
                for i in range(r):
                    k = _ldu(buf, hbase + i * 16)
                    _stu(buf, hbase + i * 16,
                         plsc.sort_key_val(k, k, descending=True)[0])
        _floop(npairs, merge)
        run *= 2


_sc_mesh = plsc.VectorSubcoreMesh(core_axis_name="c", subcore_axis_name="s")


@functools.partial(
    pl.kernel,
    out_type=jax.ShapeDtypeStruct((B, NOUT), jnp.float32),
    mesh=_sc_mesh,
    scratch_types=[
        pltpu.VMEM((VPAD,), jnp.float32),
        pltpu.VMEM((NIDX,), jnp.int32),
        pltpu.VMEM((NIDX,), jnp.int32),
        pltpu.VMEM((NIDX,), jnp.int32),
        pltpu.VMEM((NIDX,), jnp.int32),
        pltpu.VMEM((272,), jnp.int32),
        pltpu.VMEM((528,), jnp.int32),
        pltpu.VMEM((256,), jnp.int32),
        pltpu.VMEM((256,), jnp.int32),
        pltpu.VMEM((16,), jnp.int32),
        pltpu.VMEM((16,), jnp.int32),
        pltpu.VMEM((NOUT,), jnp.float32),
    ],
    compiler_params=pltpu.CompilerParams(needs_layout_passes=False),
)
def _sc_select(s_hbm, idx_hbm, out_hbm, srow_v, idx_v, keys_v, refa_v, refb_v,
               top_v, mid_v, hist_v, cum_v, h16_v, c16_v, orow_v):
    wid = lax.axis_index("s") * NC + lax.axis_index("c")
    iota = _iota16()

    def refine(dv, p):
        belowv = jnp.where(dv == 0, 0,
                           plsc.load_gather(cum_v, [jnp.maximum(dv - 1, 0)]))
        abovev = _splat(N) - plsc.load_gather(cum_v, [dv])
        plocv = _splat(p) - belowv
        prefixv = dv.astype(_U32) << 24

        def extract(j, off):
            k = _ldu(keys_v, j * 16)
            m = (k >> 24).astype(jnp.int32) == dv
            plsc.store_compressed(refa_v.at[pl.ds(off, 16)],
                                  plsc.bitcast(k, jnp.int32), mask=m)
            return off + jnp.max(_popcnt(m))
        n_sc = lax.fori_loop(0, N // 16, extract, 0)
        nv = jnp.full((16,), n_sc, jnp.int32)

        bufs = [refa_v, refb_v]
        state = (n_sc, nv, belowv, abovev, plocv, prefixv)
        for ri, shift in enumerate((20, 16, 12, 8, 4, 0)):
            src = bufs[ri % 2]
            dst = bufs[(ri + 1) % 2]
            n_sc, nv, belowv, abovev, plocv, prefixv = state
            h16_v[...] = jnp.zeros((16,), jnp.int32)
            trips = (n_sc + 15) // 16

            def hist(j, _, src=src, shift=shift, nv=nv):
                k = _ldu(src, j * 16)
                valid = (j * 16 + iota) < nv
                dd = ((k >> shift).astype(jnp.int32)) & 15
                cnt, last = plsc.scan_count(dd, mask=valid)
                plsc.addupdate_scatter(h16_v, [dd], cnt,
                                       mask=jnp.logical_and(last, valid))
                return 0
            lax.fori_loop(0, trips, hist, 0)

            c16 = plsc.cumsum(h16_v[...])
            c16_v[...] = c16
            dsv = _popcnt(c16 < (plocv + 1))
            below_inc = jnp.where(dsv == 0, 0,
                                  plsc.load_gather(c16_v, [jnp.maximum(dsv - 1, 0)]))
            le_cnt = plsc.load_gather(c16_v, [jnp.minimum(dsv, 15)])
            belowv = belowv + below_inc
            plocv = plocv - below_inc
            abovev = abovev + (nv - le_cnt)
            prefixv = prefixv | (dsv.astype(_U32) << shift)

            if shift > 0:
                def filt(j, off, src=src, dst=dst, shift=shift, nv=nv, dsv=dsv):
                    k = _ldu(src, j * 16)
                    valid = (j * 16 + iota) < nv
                    m = jnp.logical_and(
                        ((k >> shift).astype(jnp.int32) & 15) == dsv, valid)
                    plsc.store_compressed(dst.at[pl.ds(off, 16)],
                                          plsc.bitcast(k, jnp.int32), mask=m)
                    return off + jnp.max(_popcnt(m))
                n_sc = lax.fori_loop(0, trips, filt, 0)
                nv = jnp.full((16,), n_sc, jnp.int32)
            state = (n_sc, nv, belowv, abovev, plocv, prefixv)
        return state[5], state[2], state[3]

    def batch_body(bi, _carry):
        b = wid * BPW + bi
        pltpu.sync_copy(s_hbm.at[b], srow_v)
        pltpu.sync_copy(idx_hbm.at[b], idx_v)

        def gat(j):
            ii = idx_v[pl.ds(j * 16, 16)]
            _stu(keys_v, j * 16, _keyify(plsc.load_gather(srow_v, [ii])))
        _floop(NIDX // 16, gat)

        def hzero(j):
            hist_v[pl.ds(j * 16, 16)] = jnp.zeros((16,), jnp.int32)
        _floop(16, hzero)

        def hist(j):
            d = (_ldu(keys_v, j * 16) >> 24).astype(jnp.int32)
            cnt, last = plsc.scan_count(d)
            plsc.addupdate_scatter(hist_v, [d], cnt, mask=last)
        _floop(N // 16, hist)

        def cums(j, st):
            carry, d1, d2 = st
            c = plsc.cumsum(hist_v[pl.ds(j * 16, 16)]) + carry
            cum_v[pl.ds(j * 16, 16)] = c
            carry = plsc.load_gather(cum_v, [_splat(0) + (j * 16 + 15)])
            d1 = d1 + _popcnt(c < 3841)
            d2 = d2 + _popcnt(c < 512)
            return (carry, d1, d2)
        _c, d1v, d2v = lax.fori_loop(
            0, 16, cums, (_splat(0), _splat(0), _splat(0)))

        t1v, _bel1, abv1 = refine(d1v, 3840)
        t2v, bel2, _abv2 = refine(d2v, 511)
        cntgt1 = abv1
        cntge2 = _splat(N) - bel2

        def tzero(j):
            top_v[pl.ds(j * 16, 16)] = jnp.zeros((16,), jnp.int32)
        _floop(17, tzero)

        def mzero(j):
            mid_v[pl.ds(j * 16, 16)] = jnp.zeros((16,), jnp.int32)
        _floop(33, mzero)

        def compact(j, offs):
            offt, offm = offs
            k = _ldu(keys_v, j * 16)
            ki = plsc.bitcast(k, jnp.int32)
            mt = k > t1v
            mm = k < t2v
            plsc.store_compressed(top_v.at[pl.ds(offt, 16)], ki, mask=mt)
            plsc.store_compressed(mid_v.at[pl.ds(offm, 16)], ki, mask=mm)
            return (offt + jnp.max(_popcnt(mt)), offm + jnp.max(_popcnt(mm)))
        lax.fori_loop(0, N // 16, compact, (0, 0))

        _sort_desc(top_v, 16)
        _sort_desc(mid_v, 32)

        posv = _unkey(_gathu(keys_v, _splat(NIDX - 16)))
        t1f = _unkey(t1v)
        t2f = _unkey(t2v)

        def asm(j):
            gp = j * 16 + iota
            jt = jnp.clip(gp - 1, 0, 255)
            topk = _gathu(top_v, jt)
            tval = jnp.where(jt < cntgt1, _unkey(topk), t1f)
            jm = gp - 257 + _splat(3584)
            midk = _gathu(mid_v, jnp.clip(jm - cntge2, 0, 511))
            mval = jnp.where(jm < cntge2, t2f, _unkey(midk))
            val = jnp.where(gp < 1, posv, jnp.where(gp <= 256, tval, mval))
            val = jnp.where(gp > 512, jnp.zeros((16,), jnp.float32), val)
            orow_v[pl.ds(j * 16, 16)] = val
        _floop(NOUT // 16, asm)

        pltpu.sync_copy(orow_v, out_hbm.at[b])
        return _carry

    lax.fori_loop(0, BPW, batch_body, 0)


def _fin_body(sel_ref, o_ref):
    v = jnp.exp(sel_ref[...] / T)
    col = lax.broadcasted_iota(jnp.int32, (B, NOUT), 1)
    v = jnp.where(col < K + 1, v, 0.0)
    z = jnp.sum(v) * (OUT_SIZE / (B * (K + 1.0)))
    o_ref[...] = v / z


def _finalize(sel):
    return pl.pallas_call(
        _fin_body,
        out_shape=jax.ShapeDtypeStruct((B, NOUT), jnp.float32),
    )(sel)


def kernel(x, y, memory, idx):
    mem_pad = jnp.concatenate(
        [memory, jnp.zeros((VPAD - OUT_SIZE, D), jnp.float32)], axis=0)
    S = _scores(x, mem_pad)
    idx_pad = jnp.concatenate(
        [idx[:, 1:], y[:, None], jnp.zeros((B, NIDX - NDRAW), jnp.int32)],
        axis=1)
    sel = _sc_select(S, idx_pad)
    return _finalize(sel)[:, :K + 1]


# speedup vs baseline: 11.7537x; 1.3748x over previous
import functools
import jax
import jax.numpy as jnp
import numpy as np
from jax import lax
from jax.experimental import pallas as pl
from jax.experimental.pallas import tpu as pltpu
from jax.experimental.pallas import tpu_sc as plsc

B = 256
D = 128
OUT_SIZE = 100000
K = 512
T = 0.07
NDRAW = 4097
N = 4096
VPAD = 100352
NV = 2048

NC = 2
NS = 16
NW = NC * NS
BPW = B // NW
NIDX = 4112
NOUT = 528

_U32 = jnp.uint32
_TOPBIT = np.uint32(0x80000000)
_ALLBIT = np.uint32(0xFFFFFFFF)


def _mm_body(x_ref, m_ref, o_ref):
    o_ref[...] = lax.dot_general(
        x_ref[...], m_ref[...], (((1,), (1,)), ((), ())),
        preferred_element_type=jnp.float32)


def _scores(x, memory):
    return pl.pallas_call(
        _mm_body,
        grid=(VPAD // NV,),
        in_specs=[
            pl.BlockSpec((B, D), lambda i: (0, 0)),
            pl.BlockSpec((NV, D), lambda i: (i, 0)),
        ],
        out_specs=pl.BlockSpec((B, NV), lambda i: (0, i)),
        out_shape=jax.ShapeDtypeStruct((B, VPAD), jnp.float32),
    )(x, memory)


def _iota16():
    return lax.iota(jnp.int32, 16)


def _splat(v, dtype=jnp.int32):
    return jnp.full((16,), v, dtype)


def _keyify(v):
    b = plsc.bitcast(v, _U32)
    neg = b >= _TOPBIT
    return jnp.where(neg, b ^ _ALLBIT, b | _TOPBIT)


def _unkey(k):
    b = jnp.where(k >= _TOPBIT, k ^ _TOPBIT, k ^ _ALLBIT)
    return plsc.bitcast(b, jnp.float32)


def _popcnt(mask):
    return plsc.all_reduce_population_count(mask)


def _ldu(ref, start):
    return plsc.bitcast(ref[pl.ds(start, 16)], _U32)


def _stu(ref, start, v):
    ref[pl.ds(start, 16)] = plsc.bitcast(v, jnp.int32)


def _gathu(ref, idxv):
    return plsc.bitcast(plsc.load_gather(ref, [idxv]), _U32)


def _floop(n, body):
    lax.fori_loop(0, n, lambda i, c: (body(i), 0)[1], 0)


def _sort_desc(buf, nvec):
    def presort(i):
        k = _ldu(buf, i * 16)
        _stu(buf, i * 16, plsc.sort_key_val(k, k, descending=True)[0])
    _floop(nvec, presort)
    run = 1
    while run < nvec:
        npairs = nvec // (2 * run)
        r = run

        def merge(p, r=r):
            base = p * (2 * r) * 16
            for i in range(r):
                lo_i = base + i * 16
                hi_i = base + (2 * r - 1 - i) * 16
                a = _ldu(buf, lo_i)
                brev = lax.rev(_ldu(buf, hi_i), (0,))
                _stu(buf, lo_i, jnp.maximum(a, brev))
                _stu(buf, hi_i, lax.rev(jnp.minimum(a, brev), (0,)))
            for half in range(2):
                hbase = base + half * r * 16
                dist = r // 2
                while dist >= 1:
                    for g in range(0, r, 2 * dist):
                        for i in range(dist):
                            i0 = hbase + (g + i) * 16
                            i1 = hbase + (g + i + dist) * 16
                            a = _ldu(buf, i0)
                            b2 = _ldu(buf, i1)
                            _stu(buf, i0, jnp.maximum(a, b2))
                            _stu(buf, i1, jnp.minimum(a, b2))
                    dist //= 2
                for i in range(r):
                    k = _ldu(buf, hbase + i * 16)
                    _stu(buf, hbase + i * 16,
                         plsc.sort_key_val(k, k, descending=True)[0])
        _floop(npairs, merge)
        run *= 2


_sc_mesh = plsc.VectorSubcoreMesh(core_axis_name="c", subcore_axis_name="s")


@functools.partial(
    pl.kernel,
    out_type=jax.ShapeDtypeStruct((B, NOUT), jnp.float32),
    mesh=_sc_mesh,
    scratch_types=[
        pltpu.VMEM((VPAD,), jnp.float32),
        pltpu.VMEM((NIDX,), jnp.int32),
        pltpu.VMEM((NIDX,), jnp.int32),
        pltpu.VMEM((NIDX,), jnp.int32),
        pltpu.VMEM((NIDX,), jnp.int32),
        pltpu.VMEM((272,), jnp.int32),
        pltpu.VMEM((528,), jnp.int32),
        pltpu.VMEM((256,), jnp.int32),
        pltpu.VMEM((256,), jnp.int32),
        pltpu.VMEM((16,), jnp.int32),
        pltpu.VMEM((16,), jnp.int32),
        pltpu.VMEM((NIDX,), jnp.int32),
        pltpu.VMEM((NOUT,), jnp.float32),
        pltpu.SemaphoreType.DMA,
        pltpu.SemaphoreType.DMA,
    ],
    compiler_params=pltpu.CompilerParams(needs_layout_passes=False),
)
def _sc_select(s_hbm, idx_hbm, out_hbm, srow_v, idx_v, keys_v, refa_v, refb_v,
               top_v, mid_v, hist_v, cum_v, h16_v, c16_v, cand2_v, orow_v,
               sem_s, sem_i):
    wid = lax.axis_index("s") * NC + lax.axis_index("c")
    iota = _iota16()

    def refine(dv, p, cand, n_sc):
        belowv = jnp.where(dv == 0, 0,
                           plsc.load_gather(cum_v, [jnp.maximum(dv - 1, 0)]))
        abovev = _splat(N) - plsc.load_gather(cum_v, [dv])
        plocv = _splat(p) - belowv
        prefixv = dv.astype(_U32) << 24
        nv = jnp.full((16,), n_sc, jnp.int32)

        bufs = [cand, refb_v, refa_v]
        state = (n_sc, nv, belowv, abovev, plocv, prefixv)
        for ri, shift in enumerate((20, 16, 12, 8, 4, 0)):
            src = bufs[0] if ri == 0 else bufs[1 + ((ri - 1) % 2)]
            dst = bufs[1 + (ri % 2)]
            n_sc, nv, belowv, abovev, plocv, prefixv = state
            h16_v[...] = jnp.zeros((16,), jnp.int32)
            trips = (n_sc + 15) // 16

            def hist(j, _, src=src, shift=shift, nv=nv):
                k = _ldu(src, j * 16)
                valid = (j * 16 + iota) < nv
                dd = ((k >> shift).astype(jnp.int32)) & 15
                cnt, last = plsc.scan_count(dd, mask=valid)
                plsc.addupdate_scatter(h16_v, [dd], cnt,
                                       mask=jnp.logical_and(last, valid))
                return 0
            lax.fori_loop(0, trips, hist, 0)

            c16 = plsc.cumsum(h16_v[...])
            c16_v[...] = c16
            dsv = _popcnt(c16 < (plocv + 1))
            below_inc = jnp.where(dsv == 0, 0,
                                  plsc.load_gather(c16_v, [jnp.maximum(dsv - 1, 0)]))
            le_cnt = plsc.load_gather(c16_v, [jnp.minimum(dsv, 15)])
            belowv = belowv + below_inc
            plocv = plocv - below_inc
            abovev = abovev + (nv - le_cnt)
            prefixv = prefixv | (dsv.astype(_U32) << shift)

            if shift > 0:
                def filt(j, off, src=src, dst=dst, shift=shift, nv=nv, dsv=dsv):
                    k = _ldu(src, j * 16)
                    valid = (j * 16 + iota) < nv
                    m = jnp.logical_and(
                        ((k >> shift).astype(jnp.int32) & 15) == dsv, valid)
                    plsc.store_compressed(dst.at[pl.ds(off, 16)],
                                          plsc.bitcast(k, jnp.int32), mask=m)
                    return off + jnp.max(_popcnt(m))
                n_sc = lax.fori_loop(0, trips, filt, 0)
                nv = jnp.full((16,), n_sc, jnp.int32)
            state = (n_sc, nv, belowv, abovev, plocv, prefixv)
        return state[5], state[2], state[3]

    b0 = wid * BPW
    pltpu.async_copy(s_hbm.at[b0], srow_v, sem_s)
    pltpu.async_copy(idx_hbm.at[b0], idx_v, sem_i)

    def batch_body(bi, _carry):
        b = wid * BPW + bi
        pltpu.make_async_copy(s_hbm.at[b], srow_v, sem_s).wait()
        pltpu.make_async_copy(idx_hbm.at[b], idx_v, sem_i).wait()

        def hzero(j):
            hist_v[pl.ds(j * 16, 16)] = jnp.zeros((16,), jnp.int32)
        _floop(16, hzero)

        def gat_hist(j):
            ii = idx_v[pl.ds(j * 16, 16)]
            k = _keyify(plsc.load_gather(srow_v, [ii]))
            _stu(keys_v, j * 16, k)
            d = (k >> 24).astype(jnp.int32)
            cnt, last = plsc.scan_count(d)
            plsc.addupdate_scatter(hist_v, [d], cnt, mask=last)
        _floop(N // 16, gat_hist)
        ii = idx_v[pl.ds(N, 16)]
        _stu(keys_v, N, _keyify(plsc.load_gather(srow_v, [ii])))

        @pl.when(bi + 1 < BPW)
        def _prefetch():
            pltpu.async_copy(s_hbm.at[b + 1], srow_v, sem_s)
            pltpu.async_copy(idx_hbm.at[b + 1], idx_v, sem_i)

        def cums(j, st):
            carry, d1, d2 = st
            c = plsc.cumsum(hist_v[pl.ds(j * 16, 16)]) + carry
            cum_v[pl.ds(j * 16, 16)] = c
            carry = plsc.load_gather(cum_v, [_splat(0) + (j * 16 + 15)])
            d1 = d1 + _popcnt(c < 3841)
            d2 = d2 + _popcnt(c < 512)
            return (carry, d1, d2)
        _c, d1v, d2v = lax.fori_loop(
            0, 16, cums, (_splat(0), _splat(0), _splat(0)))

        def extract2(j, offs):
            o1, o2 = offs
            k = _ldu(keys_v, j * 16)
            d = (k >> 24).astype(jnp.int32)
            ki = plsc.bitcast(k, jnp.int32)
            m1 = d == d1v
            m2 = d == d2v
            plsc.store_compressed(refa_v.at[pl.ds(o1, 16)], ki, mask=m1)
            plsc.store_compressed(cand2_v.at[pl.ds(o2, 16)], ki, mask=m2)
            return (o1 + jnp.max(_popcnt(m1)), o2 + jnp.max(_popcnt(m2)))
        n1_sc, n2_sc = lax.fori_loop(0, N // 16, extract2, (0, 0))

        t1v, _bel1, abv1 = refine(d1v, 3840, refa_v, n1_sc)
        t2v, bel2, _abv2 = refine(d2v, 511, cand2_v, n2_sc)
        cntgt1 = abv1
        cntge2 = _splat(N) - bel2

        def tzero(j):
            top_v[pl.ds(j * 16, 16)] = jnp.zeros((16,), jnp.int32)
        _floop(17, tzero)

        def mzero(j):
            mid_v[pl.ds(j * 16, 16)] = jnp.zeros((16,), jnp.int32)
        _floop(33, mzero)

        def compact(j, offs):
            offt, offm = offs
            k = _ldu(keys_v, j * 16)
            ki = plsc.bitcast(k, jnp.int32)
            mt = k > t1v
            mm = k < t2v
            plsc.store_compressed(top_v.at[pl.ds(offt, 16)], ki, mask=mt)
            plsc.store_compressed(mid_v.at[pl.ds(offm, 16)], ki, mask=mm)
            return (offt + jnp.max(_popcnt(mt)), offm + jnp.max(_popcnt(mm)))
        lax.fori_loop(0, N // 16, compact, (0, 0))

        _sort_desc(top_v, 16)
        _sort_desc(mid_v, 32)

        posv = _unkey(_gathu(keys_v, _splat(NIDX - 16)))
        t1f = _unkey(t1v)
        t2f = _unkey(t2v)

        def asm(j):
            gp = j * 16 + iota
            jt = jnp.clip(gp - 1, 0, 255)
            topk = _gathu(top_v, jt)
            tval = jnp.where(jt < cntgt1, _unkey(topk), t1f)
            jm = gp - 257 + _splat(3584)
            midk = _gathu(mid_v, jnp.clip(jm - cntge2, 0, 511))
            mval = jnp.where(jm < cntge2, t2f, _unkey(midk))
            val = jnp.where(gp < 1, posv, jnp.where(gp <= 256, tval, mval))
            val = jnp.where(gp > 512, jnp.zeros((16,), jnp.float32), val)
            orow_v[pl.ds(j * 16, 16)] = val
        _floop(NOUT // 16, asm)

        pltpu.sync_copy(orow_v, out_hbm.at[b])
        return _carry

    lax.fori_loop(0, BPW, batch_body, 0)


def _fin_body(sel_ref, o_ref):
    v = jnp.exp(sel_ref[...] / T)
    col = lax.broadcasted_iota(jnp.int32, (B, NOUT), 1)
    v = jnp.where(col < K + 1, v, 0.0)
    z = jnp.sum(v) * (OUT_SIZE / (B * (K + 1.0)))
    o_ref[...] = v / z


def _finalize(sel):
    return pl.pallas_call(
        _fin_body,
        out_shape=jax.ShapeDtypeStruct((B, NOUT), jnp.float32),
    )(sel)


def kernel(x, y, memory, idx):
    S = _scores(x, memory)
    idx_pad = jnp.concatenate(
        [idx[:, 1:], y[:, None], jnp.zeros((B, NIDX - NDRAW), jnp.int32)],
        axis=1)
    sel = _sc_select(S, idx_pad)
    return _finalize(sel)[:, :K + 1]


# speedup vs baseline: 11.8491x; 1.0081x over previous
import functools
import jax
import jax.numpy as jnp
import numpy as np
from jax import lax
from jax.experimental import pallas as pl
from jax.experimental.pallas import tpu as pltpu
from jax.experimental.pallas import tpu_sc as plsc

B = 256
D = 128
OUT_SIZE = 100000
K = 512
T = 0.07
NDRAW = 4097
N = 4096
VPAD = 100352
NV = 2048

NC = 2
NS = 16
NW = NC * NS
BPW = B // NW
NIDX = 4112
NOUT = 528

_U32 = jnp.uint32
_TOPBIT = np.uint32(0x80000000)
_ALLBIT = np.uint32(0xFFFFFFFF)


def _mm_body(x_ref, m_ref, o_ref):
    o_ref[...] = lax.dot_general(
        x_ref[...], m_ref[...], (((1,), (1,)), ((), ())),
        preferred_element_type=jnp.float32)


def _scores(x, memory):
    return pl.pallas_call(
        _mm_body,
        grid=(VPAD // NV,),
        in_specs=[
            pl.BlockSpec((B, D), lambda i: (0, 0)),
            pl.BlockSpec((NV, D), lambda i: (i, 0)),
        ],
        out_specs=pl.BlockSpec((B, NV), lambda i: (0, i)),
        out_shape=jax.ShapeDtypeStruct((B, VPAD), jnp.float32),
    )(x, memory)


def _iota16():
    return lax.iota(jnp.int32, 16)


def _splat(v, dtype=jnp.int32):
    return jnp.full((16,), v, dtype)


def _keyify(v):
    b = plsc.bitcast(v, _U32)
    neg = b >= _TOPBIT
    return jnp.where(neg, b ^ _ALLBIT, b | _TOPBIT)


def _unkey(k):
    b = jnp.where(k >= _TOPBIT, k ^ _TOPBIT, k ^ _ALLBIT)
    return plsc.bitcast(b, jnp.float32)


def _popcnt(mask):
    return plsc.all_reduce_population_count(mask)


def _ldu(ref, start):
    return plsc.bitcast(ref[pl.ds(start, 16)], _U32)


def _stu(ref, start, v):
    ref[pl.ds(start, 16)] = plsc.bitcast(v, jnp.int32)


def _gathu(ref, idxv):
    return plsc.bitcast(plsc.load_gather(ref, [idxv]), _U32)


def _floop(n, body, unroll=1):
    lax.fori_loop(0, n, lambda i, c: (body(i), 0)[1], 0, unroll=unroll)


def _sort_desc(buf, nvec):
    def presort(i):
        k = _ldu(buf, i * 16)
        _stu(buf, i * 16, plsc.sort_key_val(k, k, descending=True)[0])
    _floop(nvec, presort)
    run = 1
    while run < nvec:
        npairs = nvec // (2 * run)
        r = run

        def merge(p, r=r):
            base = p * (2 * r) * 16
            for i in range(r):
                lo_i = base + i * 16
                hi_i = base + (2 * r - 1 - i) * 16
                a = _ldu(buf, lo_i)
                brev = lax.rev(_ldu(buf, hi_i), (0,))
                _stu(buf, lo_i, jnp.maximum(a, brev))
                _stu(buf, hi_i, lax.rev(jnp.minimum(a, brev), (0,)))
            for half in range(2):
                hbase = base + half * r * 16
                dist = r // 2
                while dist >= 1:
                    for g in range(0, r, 2 * dist):
                        for i in range(dist):
                            i0 = hbase + (g + i) * 16
                            i1 = hbase + (g + i + dist) * 16
                            a = _ldu(buf, i0)
                            b2 = _ldu(buf, i1)
                            _stu(buf, i0, jnp.maximum(a, b2))
                            _stu(buf, i1, jnp.minimum(a, b2))
                    dist //= 2
                for i in range(r):
                    k = _ldu(buf, hbase + i * 16)
                    _stu(buf, hbase + i * 16,
                         plsc.sort_key_val(k, k, descending=True)[0])
        _floop(npairs, merge)
        run *= 2


_sc_mesh = plsc.VectorSubcoreMesh(core_axis_name="c", subcore_axis_name="s")


@functools.partial(
    pl.kernel,
    out_type=jax.ShapeDtypeStruct((B, NOUT), jnp.float32),
    mesh=_sc_mesh,
    scratch_types=[
        pltpu.VMEM((VPAD,), jnp.float32),
        pltpu.VMEM((NIDX,), jnp.int32),
        pltpu.VMEM((NIDX,), jnp.int32),
        pltpu.VMEM((NIDX,), jnp.int32),
        pltpu.VMEM((NIDX,), jnp.int32),
        pltpu.VMEM((272,), jnp.int32),
        pltpu.VMEM((528,), jnp.int32),
        pltpu.VMEM((256,), jnp.int32),
        pltpu.VMEM((256,), jnp.int32),
        pltpu.VMEM((16,), jnp.int32),
        pltpu.VMEM((16,), jnp.int32),
        pltpu.VMEM((NIDX,), jnp.int32),
        pltpu.VMEM((NOUT,), jnp.float32),
        pltpu.SemaphoreType.DMA,
        pltpu.SemaphoreType.DMA,
    ],
    compiler_params=pltpu.CompilerParams(needs_layout_passes=False),
)
def _sc_select(s_hbm, idx_hbm, out_hbm, srow_v, idx_v, keys_v, refa_v, refb_v,
               top_v, mid_v, hist_v, cum_v, h16_v, c16_v, cand2_v, orow_v,
               sem_s, sem_i):
    wid = lax.axis_index("s") * NC + lax.axis_index("c")
    iota = _iota16()

    def refine(dv, p, cand, n_sc):
        belowv = jnp.where(dv == 0, 0,
                           plsc.load_gather(cum_v, [jnp.maximum(dv - 1, 0)]))
        abovev = _splat(N) - plsc.load_gather(cum_v, [dv])
        plocv = _splat(p) - belowv
        prefixv = dv.astype(_U32) << 24
        nv = jnp.full((16,), n_sc, jnp.int32)

        bufs = [cand, refb_v, refa_v]
        state = (n_sc, nv, belowv, abovev, plocv, prefixv)
        for ri, shift in enumerate((20, 16, 12, 8, 4, 0)):
            src = bufs[0] if ri == 0 else bufs[1 + ((ri - 1) % 2)]
            dst = bufs[1 + (ri % 2)]
            n_sc, nv, belowv, abovev, plocv, prefixv = state
            h16_v[...] = jnp.zeros((16,), jnp.int32)
            trips = (n_sc + 15) // 16

            def hist(j, _, src=src, shift=shift, nv=nv):
                k = _ldu(src, j * 16)
                valid = (j * 16 + iota) < nv
                dd = ((k >> shift).astype(jnp.int32)) & 15
                cnt, last = plsc.scan_count(dd, mask=valid)
                plsc.addupdate_scatter(h16_v, [dd], cnt,
                                       mask=jnp.logical_and(last, valid))
                return 0
            lax.fori_loop(0, trips, hist, 0)

            c16 = plsc.cumsum(h16_v[...])
            c16_v[...] = c16
            dsv = _popcnt(c16 < (plocv + 1))
            below_inc = jnp.where(dsv == 0, 0,
                                  plsc.load_gather(c16_v, [jnp.maximum(dsv - 1, 0)]))
            le_cnt = plsc.load_gather(c16_v, [jnp.minimum(dsv, 15)])
            belowv = belowv + below_inc
            plocv = plocv - below_inc
            abovev = abovev + (nv - le_cnt)
            prefixv = prefixv | (dsv.astype(_U32) << shift)

            if shift > 0:
                def filt(j, off, src=src, dst=dst, shift=shift, nv=nv, dsv=dsv):
                    k = _ldu(src, j * 16)
                    valid = (j * 16 + iota) < nv
                    m = jnp.logical_and(
                        ((k >> shift).astype(jnp.int32) & 15) == dsv, valid)
                    plsc.store_compressed(dst.at[pl.ds(off, 16)],
                                          plsc.bitcast(k, jnp.int32), mask=m)
                    return off + jnp.max(_popcnt(m))
                n_sc = lax.fori_loop(0, trips, filt, 0)
                nv = jnp.full((16,), n_sc, jnp.int32)
            state = (n_sc, nv, belowv, abovev, plocv, prefixv)
        return state[5], state[2], state[3]

    b0 = wid * BPW
    pltpu.async_copy(s_hbm.at[b0], srow_v, sem_s)
    pltpu.async_copy(idx_hbm.at[b0], idx_v, sem_i)

    def batch_body(bi, _carry):
        b = wid * BPW + bi
        pltpu.make_async_copy(s_hbm.at[b], srow_v, sem_s).wait()
        pltpu.make_async_copy(idx_hbm.at[b], idx_v, sem_i).wait()

        def hzero(j):
            hist_v[pl.ds(j * 16, 16)] = jnp.zeros((16,), jnp.int32)
        _floop(16, hzero)

        def gat_hist(j):
            ii = idx_v[pl.ds(j * 16, 16)]
            k = _keyify(plsc.load_gather(srow_v, [ii]))
            _stu(keys_v, j * 16, k)
            d = (k >> 24).astype(jnp.int32)
            cnt, last = plsc.scan_count(d)
            plsc.addupdate_scatter(hist_v, [d], cnt, mask=last)
        _floop(N // 16, gat_hist, unroll=2)
        ii = idx_v[pl.ds(N, 16)]
        _stu(keys_v, N, _keyify(plsc.load_gather(srow_v, [ii])))

        @pl.when(bi + 1 < BPW)
        def _prefetch():
            pltpu.async_copy(s_hbm.at[b + 1], srow_v, sem_s)
            pltpu.async_copy(idx_hbm.at[b + 1], idx_v, sem_i)

        def cums(j, st):
            carry, d1, d2 = st
            c = plsc.cumsum(hist_v[pl.ds(j * 16, 16)]) + carry
            cum_v[pl.ds(j * 16, 16)] = c
            carry = plsc.load_gather(cum_v, [_splat(0) + (j * 16 + 15)])
            d1 = d1 + _popcnt(c < 3841)
            d2 = d2 + _popcnt(c < 512)
            return (carry, d1, d2)
        _c, d1v, d2v = lax.fori_loop(
            0, 16, cums, (_splat(0), _splat(0), _splat(0)))

        def extract2(j, offs):
            o1, o2 = offs
            k = _ldu(keys_v, j * 16)
            d = (k >> 24).astype(jnp.int32)
            ki = plsc.bitcast(k, jnp.int32)
            m1 = d == d1v
            m2 = d == d2v
            plsc.store_compressed(refa_v.at[pl.ds(o1, 16)], ki, mask=m1)
            plsc.store_compressed(cand2_v.at[pl.ds(o2, 16)], ki, mask=m2)
            return (o1 + jnp.max(_popcnt(m1)), o2 + jnp.max(_popcnt(m2)))
        n1_sc, n2_sc = lax.fori_loop(0, N // 16, extract2, (0, 0))

        t1v, _bel1, abv1 = refine(d1v, 3840, refa_v, n1_sc)
        t2v, bel2, _abv2 = refine(d2v, 511, cand2_v, n2_sc)
        cntgt1 = abv1
        cntge2 = _splat(N) - bel2

        def tzero(j):
            top_v[pl.ds(j * 16, 16)] = jnp.zeros((16,), jnp.int32)
        _floop(17, tzero)

        def mzero(j):
            mid_v[pl.ds(j * 16, 16)] = jnp.zeros((16,), jnp.int32)
        _floop(33, mzero)

        def compact(j, offs):
            offt, offm = offs
            k = _ldu(keys_v, j * 16)
            ki = plsc.bitcast(k, jnp.int32)
            mt = k > t1v
            mm = k < t2v
            plsc.store_compressed(top_v.at[pl.ds(offt, 16)], ki, mask=mt)
            plsc.store_compressed(mid_v.at[pl.ds(offm, 16)], ki, mask=mm)
            return (offt + jnp.max(_popcnt(mt)), offm + jnp.max(_popcnt(mm)))
        lax.fori_loop(0, N // 16, compact, (0, 0))

        _sort_desc(top_v, 16)
        _sort_desc(mid_v, 32)

        posv = _unkey(_gathu(keys_v, _splat(NIDX - 16)))
        t1f = _unkey(t1v)
        t2f = _unkey(t2v)

        def asm(j):
            gp = j * 16 + iota
            jt = jnp.clip(gp - 1, 0, 255)
            topk = _gathu(top_v, jt)
            tval = jnp.where(jt < cntgt1, _unkey(topk), t1f)
            jm = gp - 257 + _splat(3584)
            midk = _gathu(mid_v, jnp.clip(jm - cntge2, 0, 511))
            mval = jnp.where(jm < cntge2, t2f, _unkey(midk))
            val = jnp.where(gp < 1, posv, jnp.where(gp <= 256, tval, mval))
            val = jnp.where(gp > 512, jnp.zeros((16,), jnp.float32), val)
            orow_v[pl.ds(j * 16, 16)] = val
        _floop(NOUT // 16, asm)

        pltpu.sync_copy(orow_v, out_hbm.at[b])
        return _carry

    lax.fori_loop(0, BPW, batch_body, 0)


def _fin_body(sel_ref, o_ref):
    v = jnp.exp(sel_ref[...] / T)
    col = lax.broadcasted_iota(jnp.int32, (B, NOUT), 1)
    v = jnp.where(col < K + 1, v, 0.0)
    z = jnp.sum(v) * (OUT_SIZE / (B * (K + 1.0)))
    o_ref[...] = v / z


def _finalize(sel):
    return pl.pallas_call(
        _fin_body,
        out_shape=jax.ShapeDtypeStruct((B, NOUT), jnp.float32),
    )(sel)


def kernel(x, y, memory, idx):
    S = _scores(x, memory)
    idx_pad = jnp.concatenate(
        [idx[:, 1:], y[:, None], jnp.zeros((B, NIDX - NDRAW), jnp.int32)],
        axis=1)
    sel = _sc_select(S, idx_pad)
    return _finalize(sel)[:, :K + 1]


# speedup vs baseline: 11.8582x; 1.0008x over previous
import functools
import jax
import jax.numpy as jnp
import numpy as np
from jax import lax
from jax.experimental import pallas as pl
from jax.experimental.pallas import tpu as pltpu
from jax.experimental.pallas import tpu_sc as plsc

B = 256
D = 128
OUT_SIZE = 100000
K = 512
T = 0.07
NDRAW = 4097
N = 4096
VPAD = 100352
NV = 2048

NC = 2
NS = 16
NW = NC * NS
BPW = B // NW
NIDX = 4112
NOUT = 528

_U32 = jnp.uint32
_TOPBIT = np.uint32(0x80000000)
_ALLBIT = np.uint32(0xFFFFFFFF)


def _mm_body(x_ref, m_ref, o_ref):
    o_ref[...] = lax.dot_general(
        x_ref[...], m_ref[...], (((1,), (1,)), ((), ())),
        preferred_element_type=jnp.float32)


def _scores(x, memory):
    return pl.pallas_call(
        _mm_body,
        grid=(VPAD // NV,),
        in_specs=[
            pl.BlockSpec((B, D), lambda i: (0, 0)),
            pl.BlockSpec((NV, D), lambda i: (i, 0)),
        ],
        out_specs=pl.BlockSpec((B, NV), lambda i: (0, i)),
        out_shape=jax.ShapeDtypeStruct((B, VPAD), jnp.float32),
    )(x, memory)


def _iota16():
    return lax.iota(jnp.int32, 16)


def _splat(v, dtype=jnp.int32):
    return jnp.full((16,), v, dtype)


def _keyify(v):
    b = plsc.bitcast(v, _U32)
    neg = b >= _TOPBIT
    return jnp.where(neg, b ^ _ALLBIT, b | _TOPBIT)


def _unkey(k):
    b = jnp.where(k >= _TOPBIT, k ^ _TOPBIT, k ^ _ALLBIT)
    return plsc.bitcast(b, jnp.float32)


def _popcnt(mask):
    return plsc.all_reduce_population_count(mask)


def _ldu(ref, start):
    return plsc.bitcast(ref[pl.ds(start, 16)], _U32)


def _stu(ref, start, v):
    ref[pl.ds(start, 16)] = plsc.bitcast(v, jnp.int32)


def _gathu(ref, idxv):
    return plsc.bitcast(plsc.load_gather(ref, [idxv]), _U32)


def _floop(n, body, unroll=1):
    lax.fori_loop(0, n, lambda i, c: (body(i), 0)[1], 0, unroll=unroll)


def _sort_desc(buf, nvec):
    def presort(i):
        k = _ldu(buf, i * 16)
        _stu(buf, i * 16, plsc.sort_key_val(k, k, descending=True)[0])
    _floop(nvec, presort)
    run = 1
    while run < nvec:
        npairs = nvec // (2 * run)
        r = run

        def merge(p, r=r):
            base = p * (2 * r) * 16
            for i in range(r):
                lo_i = base + i * 16
                hi_i = base + (2 * r - 1 - i) * 16
                a = _ldu(buf, lo_i)
                brev = lax.rev(_ldu(buf, hi_i), (0,))
                _stu(buf, lo_i, jnp.maximum(a, brev))
                _stu(buf, hi_i, lax.rev(jnp.minimum(a, brev), (0,)))
            for half in range(2):
                hbase = base + half * r * 16
                dist = r // 2
                while dist >= 1:
                    for g in range(0, r, 2 * dist):
                        for i in range(dist):
                            i0 = hbase + (g + i) * 16
                            i1 = hbase + (g + i + dist) * 16
                            a = _ldu(buf, i0)
                            b2 = _ldu(buf, i1)
                            _stu(buf, i0, jnp.maximum(a, b2))
                            _stu(buf, i1, jnp.minimum(a, b2))
                    dist //= 2
                for i in range(r):
                    k = _ldu(buf, hbase + i * 16)
                    _stu(buf, hbase + i * 16,
                         plsc.sort_key_val(k, k, descending=True)[0])
        _floop(npairs, merge)
        run *= 2


_sc_mesh = plsc.VectorSubcoreMesh(core_axis_name="c", subcore_axis_name="s")


@functools.partial(
    pl.kernel,
    out_type=jax.ShapeDtypeStruct((B, NOUT), jnp.float32),
    mesh=_sc_mesh,
    scratch_types=[
        pltpu.VMEM((VPAD,), jnp.float32),
        pltpu.VMEM((NIDX,), jnp.int32),
        pltpu.VMEM((NIDX,), jnp.int32),
        pltpu.VMEM((NIDX,), jnp.int32),
        pltpu.VMEM((NIDX,), jnp.int32),
        pltpu.VMEM((272,), jnp.int32),
        pltpu.VMEM((528,), jnp.int32),
        pltpu.VMEM((256,), jnp.int32),
        pltpu.VMEM((256,), jnp.int32),
        pltpu.VMEM((16,), jnp.int32),
        pltpu.VMEM((16,), jnp.int32),
        pltpu.VMEM((NIDX,), jnp.int32),
        pltpu.VMEM((NOUT,), jnp.float32),
        pltpu.SemaphoreType.DMA,
        pltpu.SemaphoreType.DMA,
    ],
    compiler_params=pltpu.CompilerParams(needs_layout_passes=False),
)
def _sc_select(s_hbm, idx_hbm, out_hbm, srow_v, idx_v, keys_v, refa_v, refb_v,
               top_v, mid_v, hist_v, cum_v, h16_v, c16_v, cand2_v, orow_v,
               sem_s, sem_i):
    wid = lax.axis_index("s") * NC + lax.axis_index("c")
    iota = _iota16()

    def refine(dv, p, cand, n_sc):
        belowv = jnp.where(dv == 0, 0,
                           plsc.load_gather(cum_v, [jnp.maximum(dv - 1, 0)]))
        abovev = _splat(N) - plsc.load_gather(cum_v, [dv])
        plocv = _splat(p) - belowv
        prefixv = dv.astype(_U32) << 24
        nv = jnp.full((16,), n_sc, jnp.int32)

        bufs = [cand, refb_v, refa_v]
        state = (n_sc, nv, belowv, abovev, plocv, prefixv)
        for ri, shift in enumerate((20, 16, 12, 8, 4, 0)):
            src = bufs[0] if ri == 0 else bufs[1 + ((ri - 1) % 2)]
            dst = bufs[1 + (ri % 2)]
            n_sc, nv, belowv, abovev, plocv, prefixv = state
            h16_v[...] = jnp.zeros((16,), jnp.int32)
            trips = (n_sc + 15) // 16

            def hist(j, _, src=src, shift=shift, nv=nv):
                k = _ldu(src, j * 16)
                valid = (j * 16 + iota) < nv
                dd = ((k >> shift).astype(jnp.int32)) & 15
                cnt, last = plsc.scan_count(dd, mask=valid)
                plsc.addupdate_scatter(h16_v, [dd], cnt,
                                       mask=jnp.logical_and(last, valid))
                return 0
            lax.fori_loop(0, trips, hist, 0)

            c16 = plsc.cumsum(h16_v[...])
            c16_v[...] = c16
            dsv = _popcnt(c16 < (plocv + 1))
            below_inc = jnp.where(dsv == 0, 0,
                                  plsc.load_gather(c16_v, [jnp.maximum(dsv - 1, 0)]))
            le_cnt = plsc.load_gather(c16_v, [jnp.minimum(dsv, 15)])
            belowv = belowv + below_inc
            plocv = plocv - below_inc
            abovev = abovev + (nv - le_cnt)
            prefixv = prefixv | (dsv.astype(_U32) << shift)

            if shift > 0:
                def filt(j, off, src=src, dst=dst, shift=shift, nv=nv, dsv=dsv):
                    k = _ldu(src, j * 16)
                    valid = (j * 16 + iota) < nv
                    m = jnp.logical_and(
                        ((k >> shift).astype(jnp.int32) & 15) == dsv, valid)
                    plsc.store_compressed(dst.at[pl.ds(off, 16)],
                                          plsc.bitcast(k, jnp.int32), mask=m)
                    return off + jnp.max(_popcnt(m))
                n_sc = lax.fori_loop(0, trips, filt, 0)
                nv = jnp.full((16,), n_sc, jnp.int32)
            state = (n_sc, nv, belowv, abovev, plocv, prefixv)
        return state[5], state[2], state[3]

    b0 = wid * BPW
    pltpu.async_copy(s_hbm.at[b0], srow_v, sem_s)
    pltpu.async_copy(idx_hbm.at[b0], idx_v, sem_i)

    def batch_body(bi, _carry):
        b = wid * BPW + bi
        pltpu.make_async_copy(s_hbm.at[b], srow_v, sem_s).wait()
        pltpu.make_async_copy(idx_hbm.at[b], idx_v, sem_i).wait()

        def hzero(j):
            hist_v[pl.ds(j * 16, 16)] = jnp.zeros((16,), jnp.int32)
        _floop(16, hzero)

        def gat_hist(j):
            ii = idx_v[pl.ds(j * 16, 16)]
            k = _keyify(plsc.load_gather(srow_v, [ii]))
            _stu(keys_v, j * 16, k)
            d = (k >> 24).astype(jnp.int32)
            cnt, last = plsc.scan_count(d)
            plsc.addupdate_scatter(hist_v, [d], cnt, mask=last)
        _floop(N // 16, gat_hist, unroll=2)
        ii = idx_v[pl.ds(N, 16)]
        _stu(keys_v, N, _keyify(plsc.load_gather(srow_v, [ii])))

        @pl.when(bi + 1 < BPW)
        def _prefetch():
            pltpu.async_copy(s_hbm.at[b + 1], srow_v, sem_s)
            pltpu.async_copy(idx_hbm.at[b + 1], idx_v, sem_i)

        def cums(j, st):
            carry, d1, d2 = st
            c = plsc.cumsum(hist_v[pl.ds(j * 16, 16)]) + carry
            cum_v[pl.ds(j * 16, 16)] = c
            carry = plsc.load_gather(cum_v, [_splat(0) + (j * 16 + 15)])
            d1 = d1 + _popcnt(c < 3841)
            d2 = d2 + _popcnt(c < 512)
            return (carry, d1, d2)
        _c, d1v, d2v = lax.fori_loop(
            0, 16, cums, (_splat(0), _splat(0), _splat(0)))

        def extract2(j, offs):
            o1, o2 = offs
            k = _ldu(keys_v, j * 16)
            d = (k >> 24).astype(jnp.int32)
            ki = plsc.bitcast(k, jnp.int32)
            m1 = d == d1v
            m2 = d == d2v
            plsc.store_compressed(refa_v.at[pl.ds(o1, 16)], ki, mask=m1)
            plsc.store_compressed(cand2_v.at[pl.ds(o2, 16)], ki, mask=m2)
            return (o1 + jnp.max(_popcnt(m1)), o2 + jnp.max(_popcnt(m2)))
        n1_sc, n2_sc = lax.fori_loop(0, N // 16, extract2, (0, 0))

        t1v, _bel1, abv1 = refine(d1v, 3840, refa_v, n1_sc)
        t2v, bel2, _abv2 = refine(d2v, 511, cand2_v, n2_sc)
        cntgt1 = abv1
        cntge2 = _splat(N) - bel2

        def tzero(j):
            top_v[pl.ds(j * 16, 16)] = jnp.zeros((16,), jnp.int32)
        _floop(17, tzero)

        def mzero(j):
            mid_v[pl.ds(j * 16, 16)] = jnp.zeros((16,), jnp.int32)
        _floop(33, mzero)

        def compact(j, offs):
            offt, offm = offs
            k = _ldu(keys_v, j * 16)
            ki = plsc.bitcast(k, jnp.int32)
            mt = k > t1v
            mm = k < t2v
            plsc.store_compressed(top_v.at[pl.ds(offt, 16)], ki, mask=mt)
            plsc.store_compressed(mid_v.at[pl.ds(offm, 16)], ki, mask=mm)
            return (offt + jnp.max(_popcnt(mt)), offm + jnp.max(_popcnt(mm)))
        lax.fori_loop(0, N // 16, compact, (0, 0))

        _sort_desc(top_v, 16)
        _sort_desc(mid_v, 32)

        posv = _unkey(_gathu(keys_v, _splat(NIDX - 16)))
        t1f = _unkey(t1v)
        t2f = _unkey(t2v)

        def asm(j):
            gp = j * 16 + iota
            jt = jnp.clip(gp - 1, 0, 255)
            topk = _gathu(top_v, jt)
            tval = jnp.where(jt < cntgt1, _unkey(topk), t1f)
            jm = gp - 257 + _splat(3584)
            midk = _gathu(mid_v, jnp.clip(jm - cntge2, 0, 511))
            mval = jnp.where(jm < cntge2, t2f, _unkey(midk))
            val = jnp.where(gp < 1, posv, jnp.where(gp <= 256, tval, mval))
            val = jnp.where(gp > 512, jnp.zeros((16,), jnp.float32), val)
            orow_v[pl.ds(j * 16, 16)] = val
        _floop(NOUT // 16, asm)

        pltpu.sync_copy(orow_v, out_hbm.at[b])
        return _carry

    lax.fori_loop(0, BPW, batch_body, 0)


def _fin_body(sel_ref, o_ref):
    v = jnp.exp(sel_ref[...] / T)
    col = lax.broadcasted_iota(jnp.int32, (B, NOUT), 1)
    v = jnp.where(col < K + 1, v, 0.0)
    z = jnp.sum(v) * (OUT_SIZE / (B * (K + 1.0)))
    o_ref[...] = (v / z)[:, :K + 1]


def _finalize(sel):
    return pl.pallas_call(
        _fin_body,
        out_shape=jax.ShapeDtypeStruct((B, K + 1), jnp.float32),
    )(sel)


def kernel(x, y, memory, idx):
    S = _scores(x, memory)
    idx_pad = jnp.concatenate(
        [idx[:, 1:], y[:, None], jnp.zeros((B, NIDX - NDRAW), jnp.int32)],
        axis=1)
    sel = _sc_select(S, idx_pad)
    return _finalize(sel)
